# baseline (speedup 1.0000x reference)
import jax
import jax.numpy as jnp
from jax.experimental import pallas as pl

BATCH = 4096
SEQLEN = 200
EMBED = 64
BLOCK_B = 128


def _body(x_ref, pos_ref, out_ref):
    out_ref[...] = x_ref[...][:, :, None] + pos_ref[...][None, :, :]


def kernel(x, pos_table):
    grid = (BATCH // BLOCK_B,)
    return pl.pallas_call(
        _body,
        grid=grid,
        in_specs=[
            pl.BlockSpec((BLOCK_B, SEQLEN), lambda i: (i, 0)),
            pl.BlockSpec((SEQLEN, EMBED), lambda i: (0, 0)),
        ],
        out_specs=pl.BlockSpec((BLOCK_B, SEQLEN, EMBED), lambda i: (i, 0, 0)),
        out_shape=jax.ShapeDtypeStruct((BATCH, SEQLEN, EMBED), jnp.float32),
    )(x, pos_table)
